# baseline (speedup 1.0000x reference)
import functools

import jax
import jax.numpy as jnp
from jax import lax
from jax.experimental import pallas as pl
from jax.experimental.pallas import tpu as pltpu
from jax.experimental.pallas import tpu_sc as plsc

N = 10000
E = 320000
MIN_CLIP = -5.0
MAX_CLIP = 3.0

NC = 2
NS = 16
K = 128
NI = 8
FH = 128
NPO = 10240
SROWS = NPO // NS
NB_CH0 = 20
NB_CH12 = 20


def _pipeline(srcs, dsts, table, acc, s, src_t, dst_t, rows0, rows1,
              gsem0, gsem1, nb):
    def block(sb):
        pltpu.sync_copy(srcs.at[s, sb], src_t)
        pltpu.sync_copy(dsts.at[s, sb], dst_t)
        pltpu.async_copy(table.at[src_t.at[0]], rows0, gsem0)
        for j in range(NI // 2):
            i0 = 2 * j
            i1 = i0 + 1
            pltpu.make_async_copy(table.at[src_t.at[i0]], rows0, gsem0).wait()
            pltpu.async_copy(table.at[src_t.at[i1]], rows1, gsem1)
            pltpu.sync_copy(rows0, acc.at[dst_t.at[i0]], add=True)
            pltpu.make_async_copy(table.at[src_t.at[i1]], rows1, gsem1).wait()
            if i1 + 1 < NI:
                pltpu.async_copy(table.at[src_t.at[i1 + 1]], rows0, gsem0)
            pltpu.sync_copy(rows1, acc.at[dst_t.at[i1]], add=True)

    pl.loop(0, nb)(block)


def _seg_sum_l0_body(srcs, dsts, table, zeros, out, src_t,
                     dst_t, rows0, rows1, acc, gsem0, gsem1):
    c = lax.axis_index("c")
    s = lax.axis_index("s")

    @pl.when(c == 0)
    def _():
        pltpu.sync_copy(zeros.at[pl.ds(s * SROWS, SROWS)],
                        acc.at[pl.ds(s * SROWS, SROWS)])
        plsc.subcore_barrier()
        _pipeline(srcs, dsts, table, acc, s, src_t, dst_t, rows0, rows1,
                  gsem0, gsem1, NB_CH0)
        plsc.subcore_barrier()
        pltpu.sync_copy(acc.at[pl.ds(s * SROWS, SROWS)],
                        out.at[pl.ds(s * SROWS, SROWS)])


def _seg_sum_12_body(srcs, dsts, table, zeros, out, src_t, dst_t,
                     rows0, rows1, acc, gsem0, gsem1):
    c = lax.axis_index("c")
    s = lax.axis_index("s")
    pltpu.sync_copy(zeros.at[pl.ds(s * SROWS, SROWS)],
                    acc.at[pl.ds(s * SROWS, SROWS)])
    plsc.subcore_barrier()
    _pipeline(srcs.at[c], dsts.at[c], table, acc, s, src_t, dst_t, rows0,
              rows1, gsem0, gsem1, NB_CH12)
    plsc.subcore_barrier()
    pltpu.sync_copy(acc.at[pl.ds(s * SROWS, SROWS)],
                    out.at[pl.ds(c * NPO + s * SROWS, SROWS)])


def _make_mesh():
    return plsc.VectorSubcoreMesh(core_axis_name="c", subcore_axis_name="s",
                                  num_cores=NC, num_subcores=NS)


def _scratch_types():
    return [
        pltpu.VMEM((NI, K), jnp.int32),
        pltpu.VMEM((NI, K), jnp.int32),
        pltpu.VMEM((K, FH), jnp.float32),
        pltpu.VMEM((K, FH), jnp.float32),
        pltpu.VMEM_SHARED((NPO, FH), jnp.float32),
        pltpu.SemaphoreType.DMA,
        pltpu.SemaphoreType.DMA,
    ]


def _make_seg_sum_l0():
    return pl.kernel(
        _seg_sum_l0_body,
        out_type=jax.ShapeDtypeStruct((NPO, FH), jnp.float32),
        mesh=_make_mesh(),
        scratch_types=_scratch_types(),
        name="sc_segsum_l0",
    )


def _make_seg_sum_12():
    return pl.kernel(
        _seg_sum_12_body,
        out_type=jax.ShapeDtypeStruct((2 * NPO, FH), jnp.float32),
        mesh=_make_mesh(),
        scratch_types=_scratch_types(),
        name="sc_segsum_12",
    )


BN = 80
NB = N // BN
MOFF = NPO // BN
HOFF = N // BN


def _first_body(m0, h, wn, ws, b, out):
    z = jnp.dot(m0[...], wn[...], preferred_element_type=jnp.float32)
    z += jnp.dot(h[...], ws[...], preferred_element_type=jnp.float32)
    z += b[...]
    z = jnp.maximum(z, 0.0)
    out[0] = z[:, :128]
    out[1] = z[:, 128:]


def _make_first():
    return pl.pallas_call(
        _first_body,
        grid=(NB,),
        in_specs=[
            pl.BlockSpec((BN, 128), lambda i: (i, 0)),
            pl.BlockSpec((BN, 128), lambda i: (i, 0)),
            pl.BlockSpec((128, 256), lambda i: (0, 0)),
            pl.BlockSpec((128, 256), lambda i: (0, 0)),
            pl.BlockSpec((1, 256), lambda i: (0, 0)),
        ],
        out_specs=pl.BlockSpec((2, BN, 128), lambda i: (0, i, 0)),
        out_shape=jax.ShapeDtypeStruct((2, N, 128), jnp.float32),
    )


def _halved_in_specs():
    return [
        pl.BlockSpec((BN, 128), lambda i: (i, 0)),
        pl.BlockSpec((BN, 128), lambda i: (MOFF + i, 0)),
        pl.BlockSpec((BN, 128), lambda i: (i, 0)),
        pl.BlockSpec((BN, 128), lambda i: (HOFF + i, 0)),
        pl.BlockSpec((256, 256), lambda i: (0, 0)),
        pl.BlockSpec((256, 256), lambda i: (0, 0)),
        pl.BlockSpec((1, 256), lambda i: (0, 0)),
    ]


def _mid_body(m0, m1, h0, h1, wn, ws, b, out):
    z = jnp.dot(m0[...], wn[:128, :], preferred_element_type=jnp.float32)
    z += jnp.dot(m1[...], wn[128:, :], preferred_element_type=jnp.float32)
    z += jnp.dot(h0[...], ws[:128, :], preferred_element_type=jnp.float32)
    z += jnp.dot(h1[...], ws[128:, :], preferred_element_type=jnp.float32)
    z += b[...]
    z = jnp.maximum(z, 0.0)
    out[0] = z[:, :128]
    out[1] = z[:, 128:]


def _make_mid():
    return pl.pallas_call(
        _mid_body,
        grid=(NB,),
        in_specs=_halved_in_specs(),
        out_specs=pl.BlockSpec((2, BN, 128), lambda i: (0, i, 0)),
        out_shape=jax.ShapeDtypeStruct((2, N, 128), jnp.float32),
    )


def _final_body(m0, m1, h0, h1, wn, ws, b, mean, std):
    z = jnp.dot(m0[...], wn[:128, :], preferred_element_type=jnp.float32)
    z += jnp.dot(m1[...], wn[128:, :], preferred_element_type=jnp.float32)
    z += jnp.dot(h0[...], ws[:128, :], preferred_element_type=jnp.float32)
    z += jnp.dot(h1[...], ws[128:, :], preferred_element_type=jnp.float32)
    z += b[...]
    mean[...] = z[:, :128]
    std[...] = jnp.clip(z[:, 128:], MIN_CLIP, MAX_CLIP)


def _make_final():
    return pl.pallas_call(
        _final_body,
        grid=(NB,),
        in_specs=_halved_in_specs(),
        out_specs=[pl.BlockSpec((BN, 128), lambda i: (i, 0)),
                   pl.BlockSpec((BN, 128), lambda i: (i, 0))],
        out_shape=[jax.ShapeDtypeStruct((N, 128), jnp.float32),
                   jax.ShapeDtypeStruct((N, 128), jnp.float32)],
    )


def kernel(node_features, edge_index, W_nbr_0, W_self_0, b_0, W_nbr_1,
           W_self_1, b_1, W_nbr_2, W_self_2, b_2):
    src = edge_index[0]
    dst = edge_index[1]

    ep0 = NS * NB_CH0 * NI * K
    src0 = jnp.concatenate([src, jnp.zeros((ep0 - E,), jnp.int32)]).reshape(
        NS, NB_CH0, NI, K)
    dst0 = jnp.concatenate([dst, jnp.full((ep0 - E,), N, jnp.int32)]).reshape(
        NS, NB_CH0, NI, K)

    ep = NS * NB_CH12 * NI * K
    srcp = jnp.concatenate([src, jnp.zeros((ep - E,), jnp.int32)])
    dstp = jnp.concatenate([dst, jnp.full((ep - E,), N, jnp.int32)])
    srcs12 = jnp.stack([srcp, srcp + N]).reshape(NC, NS, NB_CH12, NI, K)
    dsts12 = jnp.stack([dstp, dstp]).reshape(NC, NS, NB_CH12, NI, K)

    zeros = jnp.zeros((NPO, FH), jnp.float32)

    segsum12 = _make_seg_sum_12()
    m0 = _make_seg_sum_l0()(src0, dst0, node_features, zeros)
    h = _make_first()(m0, node_features, W_nbr_0, W_self_0,
                      b_0.reshape(1, 256)).reshape(2 * N, 128)
    mcat = segsum12(srcs12, dsts12, h, zeros)
    h = _make_mid()(mcat, mcat, h, h, W_nbr_1, W_self_1,
                    b_1.reshape(1, 256)).reshape(2 * N, 128)
    mcat = segsum12(srcs12, dsts12, h, zeros)
    mean, std = _make_final()(mcat, mcat, h, h, W_nbr_2, W_self_2,
                              b_2.reshape(1, 256))
    return (mean, std)


# speedup vs baseline: 1.0417x; 1.0417x over previous
import jax
import jax.numpy as jnp
from jax import lax
from jax.experimental import pallas as pl
from jax.experimental.pallas import tpu as pltpu
from jax.experimental.pallas import tpu_sc as plsc

N = 10000
E = 320000
MIN_CLIP = -5.0
MAX_CLIP = 3.0

NC = 2
NS = 16
K = 128
NI = 8
FH = 128
NPO = 10240
SROWS = NPO // NS
NBA = 15
NBB = 5
NB_CH12 = 20


def _pipeline(srcs, dsts, table, acc, s, src_t, dst_t, rows0, rows1,
              gsem0, gsem1, nb):
    def block(sb):
        pltpu.sync_copy(srcs.at[s, sb], src_t)
        pltpu.sync_copy(dsts.at[s, sb], dst_t)
        pltpu.async_copy(table.at[src_t.at[0]], rows0, gsem0)
        for j in range(NI // 2):
            i0 = 2 * j
            i1 = i0 + 1
            pltpu.make_async_copy(table.at[src_t.at[i0]], rows0, gsem0).wait()
            pltpu.async_copy(table.at[src_t.at[i1]], rows1, gsem1)
            pltpu.sync_copy(rows0, acc.at[dst_t.at[i0]], add=True)
            pltpu.make_async_copy(table.at[src_t.at[i1]], rows1, gsem1).wait()
            if i1 + 1 < NI:
                pltpu.async_copy(table.at[src_t.at[i1 + 1]], rows0, gsem0)
            pltpu.sync_copy(rows1, acc.at[dst_t.at[i1]], add=True)

    pl.loop(0, nb)(block)


def _seg_sum_l0_body(srcsA, dstsA, srcsB, dstsB, table, zeros, out, src_t,
                     dst_t, rows0, rows1, acc, gsem0, gsem1):
    c = lax.axis_index("c")
    s = lax.axis_index("s")
    pltpu.sync_copy(zeros.at[pl.ds(s * SROWS, SROWS)],
                    acc.at[pl.ds(s * SROWS, SROWS)])
    plsc.subcore_barrier()

    @pl.when(c == 0)
    def _():
        _pipeline(srcsA, dstsA, table, acc, s, src_t, dst_t, rows0, rows1,
                  gsem0, gsem1, NBA)

    @pl.when(c == 1)
    def _():
        _pipeline(srcsB, dstsB, table, acc, s, src_t, dst_t, rows0, rows1,
                  gsem0, gsem1, NBB)

    plsc.subcore_barrier()
    pltpu.sync_copy(acc.at[pl.ds(s * SROWS, SROWS)],
                    out.at[pl.ds(c * NPO + s * SROWS, SROWS)])


def _seg_sum_12_body(srcs, dsts, table, zeros, out, src_t, dst_t,
                     rows0, rows1, acc, gsem0, gsem1):
    c = lax.axis_index("c")
    s = lax.axis_index("s")
    pltpu.sync_copy(zeros.at[pl.ds(s * SROWS, SROWS)],
                    acc.at[pl.ds(s * SROWS, SROWS)])
    plsc.subcore_barrier()
    _pipeline(srcs.at[c], dsts.at[c], table, acc, s, src_t, dst_t, rows0,
              rows1, gsem0, gsem1, NB_CH12)
    plsc.subcore_barrier()
    pltpu.sync_copy(acc.at[pl.ds(s * SROWS, SROWS)],
                    out.at[pl.ds(c * NPO + s * SROWS, SROWS)])


def _make_mesh():
    return plsc.VectorSubcoreMesh(core_axis_name="c", subcore_axis_name="s",
                                  num_cores=NC, num_subcores=NS)


def _scratch_types():
    return [
        pltpu.VMEM((NI, K), jnp.int32),
        pltpu.VMEM((NI, K), jnp.int32),
        pltpu.VMEM((K, FH), jnp.float32),
        pltpu.VMEM((K, FH), jnp.float32),
        pltpu.VMEM_SHARED((NPO, FH), jnp.float32),
        pltpu.SemaphoreType.DMA,
        pltpu.SemaphoreType.DMA,
    ]


def _make_seg_sum_l0():
    return pl.kernel(
        _seg_sum_l0_body,
        out_type=jax.ShapeDtypeStruct((2 * NPO, FH), jnp.float32),
        mesh=_make_mesh(),
        scratch_types=_scratch_types(),
        name="sc_segsum_l0",
    )


def _make_seg_sum_12():
    return pl.kernel(
        _seg_sum_12_body,
        out_type=jax.ShapeDtypeStruct((2 * NPO, FH), jnp.float32),
        mesh=_make_mesh(),
        scratch_types=_scratch_types(),
        name="sc_segsum_12",
    )


BN = 80
NB = N // BN
MOFF = NPO // BN
HOFF = N // BN


def _pre0_body(h, ws, b, y):
    y[...] = jnp.dot(h[...], ws[...],
                     preferred_element_type=jnp.float32) + b[...]


def _make_pre0():
    return pl.pallas_call(
        _pre0_body,
        grid=(NB,),
        in_specs=[
            pl.BlockSpec((BN, 128), lambda i: (i, 0)),
            pl.BlockSpec((128, 256), lambda i: (0, 0)),
            pl.BlockSpec((1, 256), lambda i: (0, 0)),
        ],
        out_specs=pl.BlockSpec((BN, 256), lambda i: (i, 0)),
        out_shape=jax.ShapeDtypeStruct((N, 256), jnp.float32),
    )


def _pre12_body(h0, h1, ws, b, y):
    z = jnp.dot(h0[...], ws[:128, :], preferred_element_type=jnp.float32)
    z += jnp.dot(h1[...], ws[128:, :], preferred_element_type=jnp.float32)
    y[...] = z + b[...]


def _make_pre12():
    return pl.pallas_call(
        _pre12_body,
        grid=(NB,),
        in_specs=[
            pl.BlockSpec((BN, 128), lambda i: (i, 0)),
            pl.BlockSpec((BN, 128), lambda i: (HOFF + i, 0)),
            pl.BlockSpec((256, 256), lambda i: (0, 0)),
            pl.BlockSpec((1, 256), lambda i: (0, 0)),
        ],
        out_specs=pl.BlockSpec((BN, 256), lambda i: (i, 0)),
        out_shape=jax.ShapeDtypeStruct((N, 256), jnp.float32),
    )


def _comb0_body(m0, m1, wn, y, out):
    msg = m0[...] + m1[...]
    z = jnp.dot(msg, wn[...], preferred_element_type=jnp.float32) + y[...]
    z = jnp.maximum(z, 0.0)
    out[0] = z[:, :128]
    out[1] = z[:, 128:]


def _make_comb0():
    return pl.pallas_call(
        _comb0_body,
        grid=(NB,),
        in_specs=[
            pl.BlockSpec((BN, 128), lambda i: (i, 0)),
            pl.BlockSpec((BN, 128), lambda i: (MOFF + i, 0)),
            pl.BlockSpec((128, 256), lambda i: (0, 0)),
            pl.BlockSpec((BN, 256), lambda i: (i, 0)),
        ],
        out_specs=pl.BlockSpec((2, BN, 128), lambda i: (0, i, 0)),
        out_shape=jax.ShapeDtypeStruct((2, N, 128), jnp.float32),
    )


def _comb12_in_specs():
    return [
        pl.BlockSpec((BN, 128), lambda i: (i, 0)),
        pl.BlockSpec((BN, 128), lambda i: (MOFF + i, 0)),
        pl.BlockSpec((256, 256), lambda i: (0, 0)),
        pl.BlockSpec((BN, 256), lambda i: (i, 0)),
    ]


def _comb1_body(m0, m1, wn, y, out):
    z = jnp.dot(m0[...], wn[:128, :], preferred_element_type=jnp.float32)
    z += jnp.dot(m1[...], wn[128:, :], preferred_element_type=jnp.float32)
    z += y[...]
    z = jnp.maximum(z, 0.0)
    out[0] = z[:, :128]
    out[1] = z[:, 128:]


def _make_comb1():
    return pl.pallas_call(
        _comb1_body,
        grid=(NB,),
        in_specs=_comb12_in_specs(),
        out_specs=pl.BlockSpec((2, BN, 128), lambda i: (0, i, 0)),
        out_shape=jax.ShapeDtypeStruct((2, N, 128), jnp.float32),
    )


def _comb2_body(m0, m1, wn, y, mean, std):
    z = jnp.dot(m0[...], wn[:128, :], preferred_element_type=jnp.float32)
    z += jnp.dot(m1[...], wn[128:, :], preferred_element_type=jnp.float32)
    z += y[...]
    mean[...] = z[:, :128]
    std[...] = jnp.clip(z[:, 128:], MIN_CLIP, MAX_CLIP)


def _make_comb2():
    return pl.pallas_call(
        _comb2_body,
        grid=(NB,),
        in_specs=_comb12_in_specs(),
        out_specs=[pl.BlockSpec((BN, 128), lambda i: (i, 0)),
                   pl.BlockSpec((BN, 128), lambda i: (i, 0))],
        out_shape=[jax.ShapeDtypeStruct((N, 128), jnp.float32),
                   jax.ShapeDtypeStruct((N, 128), jnp.float32)],
    )


def kernel(node_features, edge_index, W_nbr_0, W_self_0, b_0, W_nbr_1,
           W_self_1, b_1, W_nbr_2, W_self_2, b_2):
    src = edge_index[0]
    dst = edge_index[1]

    ea = NS * NBA * NI * K
    eb = NS * NBB * NI * K
    srcA = src[:ea].reshape(NS, NBA, NI, K)
    dstA = dst[:ea].reshape(NS, NBA, NI, K)
    srcB = jnp.concatenate(
        [src[ea:], jnp.zeros((ea + eb - E,), jnp.int32)]).reshape(
            NS, NBB, NI, K)
    dstB = jnp.concatenate(
        [dst[ea:], jnp.full((ea + eb - E,), N, jnp.int32)]).reshape(
            NS, NBB, NI, K)

    ep = NS * NB_CH12 * NI * K
    srcp = jnp.concatenate([src, jnp.zeros((ep - E,), jnp.int32)])
    dstp = jnp.concatenate([dst, jnp.full((ep - E,), N, jnp.int32)])
    srcs12 = jnp.stack([srcp, srcp + N]).reshape(NC, NS, NB_CH12, NI, K)
    dsts12 = jnp.stack([dstp, dstp]).reshape(NC, NS, NB_CH12, NI, K)

    zeros = jnp.zeros((NPO, FH), jnp.float32)

    segsum12 = _make_seg_sum_12()
    pre12 = _make_pre12()

    mcat = _make_seg_sum_l0()(srcA, dstA, srcB, dstB, node_features, zeros)
    y0 = _make_pre0()(node_features, W_self_0, b_0.reshape(1, 256))
    h = _make_comb0()(mcat, mcat, W_nbr_0, y0).reshape(2 * N, 128)

    mcat = segsum12(srcs12, dsts12, h, zeros)
    y1 = pre12(h, h, W_self_1, b_1.reshape(1, 256))
    h = _make_comb1()(mcat, mcat, W_nbr_1, y1).reshape(2 * N, 128)

    mcat = segsum12(srcs12, dsts12, h, zeros)
    y2 = pre12(h, h, W_self_2, b_2.reshape(1, 256))
    mean, std = _make_comb2()(mcat, mcat, W_nbr_2, y2)
    return (mean, std)


# speedup vs baseline: 1.1037x; 1.0595x over previous
import jax
import jax.numpy as jnp
from jax import lax
from jax.experimental import pallas as pl
from jax.experimental.pallas import tpu as pltpu
from jax.experimental.pallas import tpu_sc as plsc

N = 10000
E = 320000
MIN_CLIP = -5.0
MAX_CLIP = 3.0

NC = 2
NS = 16
K = 128
NI = 8
FH = 128
NPO = 10240
SROWS = NPO // NS
NBA = 15
NBB = 5
NB_CH12 = 20


def _pipeline(srcs, dsts, table, acc, s, src_t, dst_t, rows0, rows1,
              gsem0, gsem1, nb):
    def block(sb):
        pltpu.sync_copy(srcs.at[s, sb], src_t)
        pltpu.sync_copy(dsts.at[s, sb], dst_t)
        pltpu.async_copy(table.at[src_t.at[0]], rows0, gsem0)
        for j in range(NI // 2):
            i0 = 2 * j
            i1 = i0 + 1
            pltpu.make_async_copy(table.at[src_t.at[i0]], rows0, gsem0).wait()
            pltpu.async_copy(table.at[src_t.at[i1]], rows1, gsem1)
            pltpu.sync_copy(rows0, acc.at[dst_t.at[i0]], add=True)
            pltpu.make_async_copy(table.at[src_t.at[i1]], rows1, gsem1).wait()
            if i1 + 1 < NI:
                pltpu.async_copy(table.at[src_t.at[i1 + 1]], rows0, gsem0)
            pltpu.sync_copy(rows1, acc.at[dst_t.at[i1]], add=True)

    pl.loop(0, nb)(block)


def _seg_sum_l0_body(srcsA, dstsA, srcsB, dstsB, table, zeros, out, src_t,
                     dst_t, rows0, rows1, acc, gsem0, gsem1):
    c = lax.axis_index("c")
    s = lax.axis_index("s")
    pltpu.sync_copy(zeros.at[pl.ds(s * SROWS, SROWS)],
                    acc.at[pl.ds(s * SROWS, SROWS)])
    plsc.subcore_barrier()

    @pl.when(c == 0)
    def _():
        _pipeline(srcsA, dstsA, table, acc, s, src_t, dst_t, rows0, rows1,
                  gsem0, gsem1, NBA)

    @pl.when(c == 1)
    def _():
        _pipeline(srcsB, dstsB, table, acc, s, src_t, dst_t, rows0, rows1,
                  gsem0, gsem1, NBB)

    plsc.subcore_barrier()
    pltpu.sync_copy(acc.at[pl.ds(s * SROWS, SROWS)],
                    out.at[pl.ds(c * NPO + s * SROWS, SROWS)])


def _seg_sum_12_body(srcs, dsts, table, zeros, out, src_t, dst_t,
                     rows0, rows1, acc, gsem0, gsem1):
    c = lax.axis_index("c")
    s = lax.axis_index("s")
    pltpu.sync_copy(zeros.at[pl.ds(s * SROWS, SROWS)],
                    acc.at[pl.ds(s * SROWS, SROWS)])
    plsc.subcore_barrier()
    _pipeline(srcs.at[c], dsts.at[c], table, acc, s, src_t, dst_t, rows0,
              rows1, gsem0, gsem1, NB_CH12)
    plsc.subcore_barrier()
    pltpu.sync_copy(acc.at[pl.ds(s * SROWS, SROWS)],
                    out.at[pl.ds(c * NPO + s * SROWS, SROWS)])


def _make_mesh():
    return plsc.VectorSubcoreMesh(core_axis_name="c", subcore_axis_name="s",
                                  num_cores=NC, num_subcores=NS)


def _scratch_types():
    return [
        pltpu.VMEM((NI, K), jnp.int32),
        pltpu.VMEM((NI, K), jnp.int32),
        pltpu.VMEM((K, FH), jnp.float32),
        pltpu.VMEM((K, FH), jnp.float32),
        pltpu.VMEM_SHARED((NPO, FH), jnp.float32),
        pltpu.SemaphoreType.DMA,
        pltpu.SemaphoreType.DMA,
    ]


def _make_seg_sum_l0():
    return pl.kernel(
        _seg_sum_l0_body,
        out_type=jax.ShapeDtypeStruct((2 * NPO, FH), jnp.float32),
        mesh=_make_mesh(),
        scratch_types=_scratch_types(),
        name="sc_segsum_l0",
    )


def _make_seg_sum_12():
    return pl.kernel(
        _seg_sum_12_body,
        out_type=jax.ShapeDtypeStruct((2 * NPO, FH), jnp.float32),
        mesh=_make_mesh(),
        scratch_types=_scratch_types(),
        name="sc_segsum_12",
    )


BN = 80
NB = N // BN
MOFF = NPO // BN
HOFF = N // BN


def _first_body(m0, m1, h, wn, ws, b, out):
    msg = m0[...] + m1[...]
    z = jnp.dot(msg, wn[...], preferred_element_type=jnp.float32)
    z += jnp.dot(h[...], ws[...], preferred_element_type=jnp.float32)
    z += b[...]
    z = jnp.maximum(z, 0.0)
    out[0] = z[:, :128]
    out[1] = z[:, 128:]


def _make_first():
    return pl.pallas_call(
        _first_body,
        grid=(NB,),
        in_specs=[
            pl.BlockSpec((BN, 128), lambda i: (i, 0)),
            pl.BlockSpec((BN, 128), lambda i: (MOFF + i, 0)),
            pl.BlockSpec((BN, 128), lambda i: (i, 0)),
            pl.BlockSpec((128, 256), lambda i: (0, 0)),
            pl.BlockSpec((128, 256), lambda i: (0, 0)),
            pl.BlockSpec((1, 256), lambda i: (0, 0)),
        ],
        out_specs=pl.BlockSpec((2, BN, 128), lambda i: (0, i, 0)),
        out_shape=jax.ShapeDtypeStruct((2, N, 128), jnp.float32),
    )


def _halved_in_specs():
    return [
        pl.BlockSpec((BN, 128), lambda i: (i, 0)),
        pl.BlockSpec((BN, 128), lambda i: (MOFF + i, 0)),
        pl.BlockSpec((BN, 128), lambda i: (i, 0)),
        pl.BlockSpec((BN, 128), lambda i: (HOFF + i, 0)),
        pl.BlockSpec((256, 256), lambda i: (0, 0)),
        pl.BlockSpec((256, 256), lambda i: (0, 0)),
        pl.BlockSpec((1, 256), lambda i: (0, 0)),
    ]


def _mid_body(m0, m1, h0, h1, wn, ws, b, out):
    z = jnp.dot(m0[...], wn[:128, :], preferred_element_type=jnp.float32)
    z += jnp.dot(m1[...], wn[128:, :], preferred_element_type=jnp.float32)
    z += jnp.dot(h0[...], ws[:128, :], preferred_element_type=jnp.float32)
    z += jnp.dot(h1[...], ws[128:, :], preferred_element_type=jnp.float32)
    z += b[...]
    z = jnp.maximum(z, 0.0)
    out[0] = z[:, :128]
    out[1] = z[:, 128:]


def _make_mid():
    return pl.pallas_call(
        _mid_body,
        grid=(NB,),
        in_specs=_halved_in_specs(),
        out_specs=pl.BlockSpec((2, BN, 128), lambda i: (0, i, 0)),
        out_shape=jax.ShapeDtypeStruct((2, N, 128), jnp.float32),
    )


def _final_body(m0, m1, h0, h1, wn, ws, b, mean, std):
    z = jnp.dot(m0[...], wn[:128, :], preferred_element_type=jnp.float32)
    z += jnp.dot(m1[...], wn[128:, :], preferred_element_type=jnp.float32)
    z += jnp.dot(h0[...], ws[:128, :], preferred_element_type=jnp.float32)
    z += jnp.dot(h1[...], ws[128:, :], preferred_element_type=jnp.float32)
    z += b[...]
    mean[...] = z[:, :128]
    std[...] = jnp.clip(z[:, 128:], MIN_CLIP, MAX_CLIP)


def _make_final():
    return pl.pallas_call(
        _final_body,
        grid=(NB,),
        in_specs=_halved_in_specs(),
        out_specs=[pl.BlockSpec((BN, 128), lambda i: (i, 0)),
                   pl.BlockSpec((BN, 128), lambda i: (i, 0))],
        out_shape=[jax.ShapeDtypeStruct((N, 128), jnp.float32),
                   jax.ShapeDtypeStruct((N, 128), jnp.float32)],
    )


def kernel(node_features, edge_index, W_nbr_0, W_self_0, b_0, W_nbr_1,
           W_self_1, b_1, W_nbr_2, W_self_2, b_2):
    src = edge_index[0]
    dst = edge_index[1]

    ea = NS * NBA * NI * K
    eb = NS * NBB * NI * K
    srcA = src[:ea].reshape(NS, NBA, NI, K)
    dstA = dst[:ea].reshape(NS, NBA, NI, K)
    srcB = jnp.concatenate(
        [src[ea:], jnp.zeros((ea + eb - E,), jnp.int32)]).reshape(
            NS, NBB, NI, K)
    dstB = jnp.concatenate(
        [dst[ea:], jnp.full((ea + eb - E,), N, jnp.int32)]).reshape(
            NS, NBB, NI, K)

    ep = NS * NB_CH12 * NI * K
    srcp = jnp.concatenate([src, jnp.zeros((ep - E,), jnp.int32)])
    dstp = jnp.concatenate([dst, jnp.full((ep - E,), N, jnp.int32)])
    srcs12 = jnp.stack([srcp, srcp + N]).reshape(NC, NS, NB_CH12, NI, K)
    dsts12 = jnp.stack([dstp, dstp]).reshape(NC, NS, NB_CH12, NI, K)

    zeros = jnp.zeros((NPO, FH), jnp.float32)

    segsum12 = _make_seg_sum_12()
    mcat = _make_seg_sum_l0()(srcA, dstA, srcB, dstB, node_features, zeros)
    h = _make_first()(mcat, mcat, node_features, W_nbr_0, W_self_0,
                      b_0.reshape(1, 256)).reshape(2 * N, 128)
    mcat = segsum12(srcs12, dsts12, h, zeros)
    h = _make_mid()(mcat, mcat, h, h, W_nbr_1, W_self_1,
                    b_1.reshape(1, 256)).reshape(2 * N, 128)
    mcat = segsum12(srcs12, dsts12, h, zeros)
    mean, std = _make_final()(mcat, mcat, h, h, W_nbr_2, W_self_2,
                              b_2.reshape(1, 256))
    return (mean, std)
